# Initial kernel scaffold; baseline (speedup 1.0000x reference)
import jax
import jax.numpy as jnp
from jax.experimental import pallas as pl


def kernel(x, edge_index, batch, W1, b1, W2, b2, W3, b3, Wm1, bm1, Wm2, bm2):
    raise NotImplementedError("write your pallas kernel here")


# speedup vs baseline: 5.0277x; 5.0277x over previous
import functools

import jax
import jax.numpy as jnp
from jax import lax
from jax.experimental import pallas as pl
from jax.experimental.pallas import tpu as pltpu
from jax.experimental.pallas import tpu_sc as plsc

N = 10000
NP = 10240
E = 160000
NUM_GRAPHS = 64
HID = 512
NB = 4
BW = 128
NTILES = 16
ROWS_PER_TILE = NP // NTILES
CHUNK = 128
NCHUNK = 80
EP = NTILES * NCHUNK * CHUNK
DUMMY = NP - 8

@functools.cache
def _mesh():
    return plsc.VectorSubcoreMesh(core_axis_name="c", subcore_axis_name="s",
                                  num_cores=2, num_subcores=NTILES)


def _deg_body(dst_hbm, ones_hbm, deg_hbm, accd, dst_v, ones_v):
    c = lax.axis_index("c")
    t = lax.axis_index("s")
    pltpu.sync_copy(dst_hbm.at[t], dst_v)
    pltpu.sync_copy(ones_hbm, ones_v)
    for k in range(ROWS_PER_TILE // CHUNK):
        pltpu.sync_copy(
            ones_v, accd.at[pl.ds(t * ROWS_PER_TILE + k * CHUNK, CHUNK)])
    plsc.subcore_barrier()

    @pl.loop(0, NCHUNK)
    def _(j):
        pltpu.sync_copy(ones_v, accd.at[dst_v.at[j]], add=True)

    plsc.subcore_barrier()
    pltpu.sync_copy(accd.at[pl.ds(t * ROWS_PER_TILE, ROWS_PER_TILE)],
                    deg_hbm.at[c, pl.ds(t * ROWS_PER_TILE, ROWS_PER_TILE)])


@jax.jit
def _deg_call(dstp, ones):
    k = pl.kernel(
        _deg_body,
        out_type=jax.ShapeDtypeStruct((2, NP, BW), jnp.float32),
        mesh=_mesh(),
        scratch_types=[
            pltpu.VMEM_SHARED((NP, BW), jnp.float32),
            pltpu.VMEM((NCHUNK, CHUNK), jnp.int32),
            pltpu.VMEM((CHUNK, BW), jnp.float32),
        ],
    )
    return k(dstp, ones)


def _gs_body(h_hbm, src_hbm, dst_hbm, out_hbm, acc, src_v, dst_v, g0):
    c = lax.axis_index("c")
    t = lax.axis_index("s")
    pltpu.sync_copy(dst_hbm.at[t], dst_v)
    for b in range(2):
        blk = c * 2 + b
        pltpu.sync_copy(
            h_hbm.at[pl.ds(blk * NP + t * ROWS_PER_TILE, ROWS_PER_TILE)],
            acc.at[pl.ds(t * ROWS_PER_TILE, ROWS_PER_TILE)])
        pltpu.sync_copy(src_hbm.at[blk * NTILES + t], src_v)
        plsc.subcore_barrier()

        @pl.loop(0, NCHUNK)
        def _(j):
            pltpu.sync_copy(h_hbm.at[src_v.at[j]], g0)
            pltpu.sync_copy(g0, acc.at[dst_v.at[j]], add=True)

        plsc.subcore_barrier()
        pltpu.sync_copy(
            acc.at[pl.ds(t * ROWS_PER_TILE, ROWS_PER_TILE)],
            out_hbm.at[pl.ds(blk * NP + t * ROWS_PER_TILE, ROWS_PER_TILE)])
        plsc.subcore_barrier()


@jax.jit
def _gs_call(h, srcp_off, dstp):
    k = pl.kernel(
        _gs_body,
        out_type=jax.ShapeDtypeStruct((NB * NP, BW), jnp.float32),
        mesh=_mesh(),
        scratch_types=[
            pltpu.VMEM_SHARED((NP, BW), jnp.float32),
            pltpu.VMEM((NCHUNK, CHUNK), jnp.int32),
            pltpu.VMEM((NCHUNK, CHUNK), jnp.int32),
            pltpu.VMEM((CHUNK, BW), jnp.float32),
        ],
    )
    return k(h, srcp_off, dstp)


_TC_ROWS = 512
_GRID = NP // _TC_ROWS


def _dinv(deg_ref):
    return lax.rsqrt(deg_ref[0][:, 0:1])


def _mm1_body(x_ref, w_ref, deg_ref, out_ref):
    dinv = _dinv(deg_ref)
    h = jnp.dot(x_ref[...], w_ref[...], preferred_element_type=jnp.float32)
    hp = h * dinv
    for b in range(NB):
        out_ref[b] = hp[:, b * BW:(b + 1) * BW]


@jax.jit
def _mm1_call(x_p, W1, deg):
    in_dim = x_p.shape[1]
    return pl.pallas_call(
        _mm1_body,
        grid=(_GRID,),
        in_specs=[
            pl.BlockSpec((_TC_ROWS, in_dim), lambda i: (i, 0)),
            pl.BlockSpec((in_dim, HID), lambda i: (0, 0)),
            pl.BlockSpec((2, _TC_ROWS, BW), lambda i: (0, i, 0)),
        ],
        out_specs=pl.BlockSpec((NB, _TC_ROWS, BW), lambda i: (0, i, 0)),
        out_shape=jax.ShapeDtypeStruct((NB, NP, BW), jnp.float32),
        compiler_params=pltpu.CompilerParams(
            dimension_semantics=("parallel",)),
    )(x_p, W1, deg)


def _fmm_body(acc_ref, deg_ref, bias_ref, w_ref, out_ref):
    dinv = _dinv(deg_ref)
    a = jnp.concatenate([acc_ref[b] for b in range(NB)], axis=1)
    y = jnp.maximum(a * dinv + bias_ref[...], 0.0)
    h = jnp.dot(y, w_ref[...], preferred_element_type=jnp.float32)
    hp = h * dinv
    for b in range(NB):
        out_ref[b] = hp[:, b * BW:(b + 1) * BW]


@jax.jit
def _fmm_call(a, deg, bias, W):
    return pl.pallas_call(
        _fmm_body,
        grid=(_GRID,),
        in_specs=[
            pl.BlockSpec((NB, _TC_ROWS, BW), lambda i: (0, i, 0)),
            pl.BlockSpec((2, _TC_ROWS, BW), lambda i: (0, i, 0)),
            pl.BlockSpec((1, HID), lambda i: (0, 0)),
            pl.BlockSpec((HID, HID), lambda i: (0, 0)),
        ],
        out_specs=pl.BlockSpec((NB, _TC_ROWS, BW), lambda i: (0, i, 0)),
        out_shape=jax.ShapeDtypeStruct((NB, NP, BW), jnp.float32),
        compiler_params=pltpu.CompilerParams(
            dimension_semantics=("parallel",)),
    )(a, deg, bias, W)


def _pool_body(acc_ref, deg_ref, bias_ref, batch_ref, wm1_ref, bm1_ref,
               wm2_ref, bm2_ref, out_ref, g_acc, c_acc):
    i = pl.program_id(0)

    @pl.when(i == 0)
    def _():
        g_acc[...] = jnp.zeros_like(g_acc)
        c_acc[...] = jnp.zeros_like(c_acc)

    dinv = _dinv(deg_ref)
    a = jnp.concatenate([acc_ref[b] for b in range(NB)], axis=1)
    y = jnp.maximum(a * dinv + bias_ref[...], 0.0)
    bt = jnp.broadcast_to(batch_ref[...], (NUM_GRAPHS, _TC_ROWS))
    pt = (bt == lax.broadcasted_iota(jnp.int32, (NUM_GRAPHS, _TC_ROWS), 0)
          ).astype(jnp.float32)
    g_acc[...] += jnp.dot(pt, y, preferred_element_type=jnp.float32)
    c_acc[...] += jnp.dot(pt, jnp.ones((_TC_ROWS, 1), jnp.float32),
                          preferred_element_type=jnp.float32)

    @pl.when(i == pl.num_programs(0) - 1)
    def _():
        g = g_acc[...] / jnp.maximum(c_acc[...], 1.0)
        h1 = jnp.maximum(
            jnp.dot(g, wm1_ref[...], preferred_element_type=jnp.float32)
            + bm1_ref[...], 0.0)
        out_ref[...] = (
            jnp.dot(h1, wm2_ref[...], preferred_element_type=jnp.float32)
            + bm2_ref[...])


@jax.jit
def _pool_call(a, deg, bias, batch_p, Wm1, bm1, Wm2, bm2):
    return pl.pallas_call(
        _pool_body,
        grid=(_GRID,),
        in_specs=[
            pl.BlockSpec((NB, _TC_ROWS, BW), lambda i: (0, i, 0)),
            pl.BlockSpec((2, _TC_ROWS, BW), lambda i: (0, i, 0)),
            pl.BlockSpec((1, HID), lambda i: (0, 0)),
            pl.BlockSpec((1, _TC_ROWS), lambda i: (0, i)),
            pl.BlockSpec((HID, HID), lambda i: (0, 0)),
            pl.BlockSpec((1, HID), lambda i: (0, 0)),
            pl.BlockSpec((HID, 1), lambda i: (0, 0)),
            pl.BlockSpec((1, 1), lambda i: (0, 0)),
        ],
        out_specs=pl.BlockSpec((NUM_GRAPHS, 1), lambda i: (0, 0)),
        out_shape=jax.ShapeDtypeStruct((NUM_GRAPHS, 1), jnp.float32),
        scratch_shapes=[
            pltpu.VMEM((NUM_GRAPHS, HID), jnp.float32),
            pltpu.VMEM((NUM_GRAPHS, 1), jnp.float32),
        ],
        compiler_params=pltpu.CompilerParams(
            dimension_semantics=("arbitrary",)),
    )(a, deg, bias, batch_p, Wm1, bm1, Wm2, bm2)


def kernel(x, edge_index, batch, W1, b1, W2, b2, W3, b3, Wm1, bm1, Wm2, bm2):
    x = x.astype(jnp.float32)
    src = edge_index[0].astype(jnp.int32)
    dst = edge_index[1].astype(jnp.int32)
    pad = EP - E
    srcp = jnp.concatenate(
        [src, jnp.zeros((pad,), jnp.int32)]).reshape(NTILES, NCHUNK, CHUNK)
    srcp_off = jnp.stack(
        [srcp + blk * NP for blk in range(NB)]).reshape(
            NB * NTILES, NCHUNK, CHUNK)
    dstp = jnp.concatenate(
        [dst, jnp.full((pad,), DUMMY, jnp.int32)]).reshape(
            NTILES, NCHUNK, CHUNK)
    x_p = jnp.pad(x, ((0, NP - N), (0, 0)))
    batch_p = jnp.pad(batch.astype(jnp.int32), (0, NP - N),
                      constant_values=NUM_GRAPHS).reshape(1, NP)
    ones = jnp.ones((CHUNK, BW), jnp.float32)

    deg = _deg_call(dstp, ones)
    h1 = _mm1_call(x_p, W1, deg)
    a1 = _gs_call(h1.reshape(NB * NP, BW), srcp_off, dstp).reshape(NB, NP, BW)
    h2 = _fmm_call(a1, deg, b1.reshape(1, HID), W2)
    a2 = _gs_call(h2.reshape(NB * NP, BW), srcp_off, dstp).reshape(NB, NP, BW)
    h3 = _fmm_call(a2, deg, b2.reshape(1, HID), W3)
    a3 = _gs_call(h3.reshape(NB * NP, BW), srcp_off, dstp).reshape(NB, NP, BW)
    out = _pool_call(a3, deg, b3.reshape(1, HID), batch_p, Wm1,
                     bm1.reshape(1, HID), Wm2, bm2.reshape(1, 1))
    return out.reshape(-1)


# speedup vs baseline: 5.7491x; 1.1435x over previous
import functools

import jax
import jax.numpy as jnp
from jax import lax
from jax.experimental import pallas as pl
from jax.experimental.pallas import tpu as pltpu
from jax.experimental.pallas import tpu_sc as plsc

N = 10000
NP = 10240
E = 160000
NUM_GRAPHS = 64
HID = 512
NB = 4
BW = 128
NTILES = 16
ROWS_PER_TILE = NP // NTILES
CHUNK = 128
NCHUNK = 80
NHALF = 2
HALF = NCHUNK // NHALF
EP = NTILES * NCHUNK * CHUNK
DUMMY = NP - 8

@functools.cache
def _mesh():
    return plsc.VectorSubcoreMesh(core_axis_name="c", subcore_axis_name="s",
                                  num_cores=2, num_subcores=NTILES)


def _deg_body(dst_hbm, ones_hbm, deg_hbm, accd, dst_v, ones_v):
    c = lax.axis_index("c")
    t = lax.axis_index("s")
    pltpu.sync_copy(ones_hbm, ones_v)
    for k in range(ROWS_PER_TILE // CHUNK):
        pltpu.sync_copy(
            ones_v, accd.at[pl.ds(t * ROWS_PER_TILE + k * CHUNK, CHUNK)])
    plsc.subcore_barrier()

    for hh in range(NHALF):
        pltpu.sync_copy(dst_hbm.at[t * NHALF + hh], dst_v)

        @pl.loop(0, HALF)
        def _(j):
            pltpu.sync_copy(ones_v, accd.at[dst_v.at[j]], add=True)

    plsc.subcore_barrier()
    pltpu.sync_copy(accd.at[pl.ds(t * ROWS_PER_TILE, ROWS_PER_TILE)],
                    deg_hbm.at[c, pl.ds(t * ROWS_PER_TILE, ROWS_PER_TILE)])


@jax.jit
def _deg_call(dstp, ones):
    k = pl.kernel(
        _deg_body,
        out_type=jax.ShapeDtypeStruct((2, NP, BW), jnp.float32),
        mesh=_mesh(),
        scratch_types=[
            pltpu.VMEM_SHARED((NP, BW), jnp.float32),
            pltpu.VMEM((HALF, CHUNK), jnp.int32),
            pltpu.VMEM((CHUNK, BW), jnp.float32),
        ],
    )
    return k(dstp, ones)


_NBUF = 2


def _gs_body(h_hbm, src_hbm, dst_hbm, out_hbm, acc, src_v, dst_v,
             g0, g1, s0, s1):
    c = lax.axis_index("c")
    t = lax.axis_index("s")
    bufs = (g0, g1)
    sems = (s0, s1)
    for b in range(2):
        blk = c * 2 + b
        pltpu.sync_copy(
            h_hbm.at[pl.ds(blk * NP + t * ROWS_PER_TILE, ROWS_PER_TILE)],
            acc.at[pl.ds(t * ROWS_PER_TILE, ROWS_PER_TILE)])
        plsc.subcore_barrier()

        for hh in range(NHALF):
            pltpu.sync_copy(
                src_hbm.at[(blk * NTILES + t) * NHALF + hh], src_v)
            pltpu.sync_copy(dst_hbm.at[t * NHALF + hh], dst_v)

            for i in range(_NBUF):
                pltpu.async_copy(h_hbm.at[src_v.at[i]], bufs[i], sems[i])

            @pl.loop(0, HALF, step=_NBUF)
            def _(j):
                for i in range(_NBUF):
                    pltpu.make_async_copy(
                        h_hbm.at[pl.ds(0, CHUNK)], bufs[i], sems[i]).wait()
                    pltpu.sync_copy(bufs[i], acc.at[dst_v.at[j + i]],
                                    add=True)
                    nxt = jnp.minimum(j + i + _NBUF, HALF - 1)
                    pltpu.async_copy(h_hbm.at[src_v.at[nxt]], bufs[i],
                                     sems[i])

            for i in range(_NBUF):
                pltpu.make_async_copy(
                    h_hbm.at[pl.ds(0, CHUNK)], bufs[i], sems[i]).wait()

        plsc.subcore_barrier()
        pltpu.sync_copy(
            acc.at[pl.ds(t * ROWS_PER_TILE, ROWS_PER_TILE)],
            out_hbm.at[pl.ds(blk * NP + t * ROWS_PER_TILE, ROWS_PER_TILE)])
        plsc.subcore_barrier()


@jax.jit
def _gs_call(h, srcp_off, dstp):
    k = pl.kernel(
        _gs_body,
        out_type=jax.ShapeDtypeStruct((NB * NP, BW), jnp.float32),
        mesh=_mesh(),
        scratch_types=[
            pltpu.VMEM_SHARED((NP, BW), jnp.float32),
            pltpu.VMEM((HALF, CHUNK), jnp.int32),
            pltpu.VMEM((HALF, CHUNK), jnp.int32),
            pltpu.VMEM((CHUNK, BW), jnp.float32),
            pltpu.VMEM((CHUNK, BW), jnp.float32),
            pltpu.SemaphoreType.DMA,
            pltpu.SemaphoreType.DMA,
        ],
    )
    return k(h, srcp_off, dstp)


_TC_ROWS = 512
_GRID = NP // _TC_ROWS


def _dinv(deg_ref):
    return lax.rsqrt(deg_ref[0][:, 0:1])


def _mm1_body(x_ref, w_ref, deg_ref, out_ref):
    dinv = _dinv(deg_ref)
    h = jnp.dot(x_ref[...], w_ref[...], preferred_element_type=jnp.float32)
    hp = h * dinv
    for b in range(NB):
        out_ref[b] = hp[:, b * BW:(b + 1) * BW]


@jax.jit
def _mm1_call(x_p, W1, deg):
    in_dim = x_p.shape[1]
    return pl.pallas_call(
        _mm1_body,
        grid=(_GRID,),
        in_specs=[
            pl.BlockSpec((_TC_ROWS, in_dim), lambda i: (i, 0)),
            pl.BlockSpec((in_dim, HID), lambda i: (0, 0)),
            pl.BlockSpec((2, _TC_ROWS, BW), lambda i: (0, i, 0)),
        ],
        out_specs=pl.BlockSpec((NB, _TC_ROWS, BW), lambda i: (0, i, 0)),
        out_shape=jax.ShapeDtypeStruct((NB, NP, BW), jnp.float32),
        compiler_params=pltpu.CompilerParams(
            dimension_semantics=("parallel",)),
    )(x_p, W1, deg)


def _fmm_body(acc_ref, deg_ref, bias_ref, w_ref, out_ref):
    dinv = _dinv(deg_ref)
    a = jnp.concatenate([acc_ref[b] for b in range(NB)], axis=1)
    y = jnp.maximum(a * dinv + bias_ref[...], 0.0)
    h = jnp.dot(y, w_ref[...], preferred_element_type=jnp.float32)
    hp = h * dinv
    for b in range(NB):
        out_ref[b] = hp[:, b * BW:(b + 1) * BW]


@jax.jit
def _fmm_call(a, deg, bias, W):
    return pl.pallas_call(
        _fmm_body,
        grid=(_GRID,),
        in_specs=[
            pl.BlockSpec((NB, _TC_ROWS, BW), lambda i: (0, i, 0)),
            pl.BlockSpec((2, _TC_ROWS, BW), lambda i: (0, i, 0)),
            pl.BlockSpec((1, HID), lambda i: (0, 0)),
            pl.BlockSpec((HID, HID), lambda i: (0, 0)),
        ],
        out_specs=pl.BlockSpec((NB, _TC_ROWS, BW), lambda i: (0, i, 0)),
        out_shape=jax.ShapeDtypeStruct((NB, NP, BW), jnp.float32),
        compiler_params=pltpu.CompilerParams(
            dimension_semantics=("parallel",)),
    )(a, deg, bias, W)


def _pool_body(acc_ref, deg_ref, bias_ref, batch_ref, wm1_ref, bm1_ref,
               wm2_ref, bm2_ref, out_ref, g_acc, c_acc):
    i = pl.program_id(0)

    @pl.when(i == 0)
    def _():
        g_acc[...] = jnp.zeros_like(g_acc)
        c_acc[...] = jnp.zeros_like(c_acc)

    dinv = _dinv(deg_ref)
    a = jnp.concatenate([acc_ref[b] for b in range(NB)], axis=1)
    y = jnp.maximum(a * dinv + bias_ref[...], 0.0)
    bt = jnp.broadcast_to(batch_ref[...], (NUM_GRAPHS, _TC_ROWS))
    pt = (bt == lax.broadcasted_iota(jnp.int32, (NUM_GRAPHS, _TC_ROWS), 0)
          ).astype(jnp.float32)
    g_acc[...] += jnp.dot(pt, y, preferred_element_type=jnp.float32)
    c_acc[...] += jnp.dot(pt, jnp.ones((_TC_ROWS, 1), jnp.float32),
                          preferred_element_type=jnp.float32)

    @pl.when(i == pl.num_programs(0) - 1)
    def _():
        g = g_acc[...] / jnp.maximum(c_acc[...], 1.0)
        h1 = jnp.maximum(
            jnp.dot(g, wm1_ref[...], preferred_element_type=jnp.float32)
            + bm1_ref[...], 0.0)
        out_ref[...] = (
            jnp.dot(h1, wm2_ref[...], preferred_element_type=jnp.float32)
            + bm2_ref[...])


@jax.jit
def _pool_call(a, deg, bias, batch_p, Wm1, bm1, Wm2, bm2):
    return pl.pallas_call(
        _pool_body,
        grid=(_GRID,),
        in_specs=[
            pl.BlockSpec((NB, _TC_ROWS, BW), lambda i: (0, i, 0)),
            pl.BlockSpec((2, _TC_ROWS, BW), lambda i: (0, i, 0)),
            pl.BlockSpec((1, HID), lambda i: (0, 0)),
            pl.BlockSpec((1, _TC_ROWS), lambda i: (0, i)),
            pl.BlockSpec((HID, HID), lambda i: (0, 0)),
            pl.BlockSpec((1, HID), lambda i: (0, 0)),
            pl.BlockSpec((HID, 1), lambda i: (0, 0)),
            pl.BlockSpec((1, 1), lambda i: (0, 0)),
        ],
        out_specs=pl.BlockSpec((NUM_GRAPHS, 1), lambda i: (0, 0)),
        out_shape=jax.ShapeDtypeStruct((NUM_GRAPHS, 1), jnp.float32),
        scratch_shapes=[
            pltpu.VMEM((NUM_GRAPHS, HID), jnp.float32),
            pltpu.VMEM((NUM_GRAPHS, 1), jnp.float32),
        ],
        compiler_params=pltpu.CompilerParams(
            dimension_semantics=("arbitrary",)),
    )(a, deg, bias, batch_p, Wm1, bm1, Wm2, bm2)


def kernel(x, edge_index, batch, W1, b1, W2, b2, W3, b3, Wm1, bm1, Wm2, bm2):
    x = x.astype(jnp.float32)
    src = edge_index[0].astype(jnp.int32)
    dst = edge_index[1].astype(jnp.int32)
    pad = EP - E
    srcp = jnp.concatenate(
        [src, jnp.zeros((pad,), jnp.int32)]).reshape(NTILES, NCHUNK, CHUNK)
    srcp_off = jnp.stack(
        [srcp + blk * NP for blk in range(NB)]).reshape(
            NB * NTILES * NHALF, HALF, CHUNK)
    dstp = jnp.concatenate(
        [dst, jnp.full((pad,), DUMMY, jnp.int32)]).reshape(
            NTILES * NHALF, HALF, CHUNK)
    x_p = jnp.pad(x, ((0, NP - N), (0, 0)))
    batch_p = jnp.pad(batch.astype(jnp.int32), (0, NP - N),
                      constant_values=NUM_GRAPHS).reshape(1, NP)
    ones = jnp.ones((CHUNK, BW), jnp.float32)

    deg = _deg_call(dstp, ones)
    h1 = _mm1_call(x_p, W1, deg)
    a1 = _gs_call(h1.reshape(NB * NP, BW), srcp_off, dstp).reshape(NB, NP, BW)
    h2 = _fmm_call(a1, deg, b1.reshape(1, HID), W2)
    a2 = _gs_call(h2.reshape(NB * NP, BW), srcp_off, dstp).reshape(NB, NP, BW)
    h3 = _fmm_call(a2, deg, b2.reshape(1, HID), W3)
    a3 = _gs_call(h3.reshape(NB * NP, BW), srcp_off, dstp).reshape(NB, NP, BW)
    out = _pool_call(a3, deg, b3.reshape(1, HID), batch_p, Wm1,
                     bm1.reshape(1, HID), Wm2, bm2.reshape(1, 1))
    return out.reshape(-1)
